# baseline (speedup 1.0000x reference)
import functools

import jax
import jax.numpy as jnp
from jax import lax
from jax.experimental import pallas as pl
from jax.experimental.pallas import tpu as pltpu
from jax.experimental.pallas import tpu_sc as plsc

N = 10000
NPAD = 10240
E = 320000
ED = 640000
NC = 2
NS = 16
NW = NC * NS
CH = 250
NROWS = E // CH
CPW = NROWS // NW
RPT = NPAD // NS

EPW = ED // NW
DCH = 2000
NG = DCH // 16
NBUF = 4


def _make_seg_kernel(D):
    mesh = plsc.VectorSubcoreMesh(core_axis_name="c", subcore_axis_name="s")

    @functools.partial(
        pl.kernel,
        mesh=mesh,
        out_type=jax.ShapeDtypeStruct((NC * NPAD, D), jnp.float32),
        compiler_params=pltpu.CompilerParams(use_tc_tiling_on_sc=False),
        scratch_types=[
            pltpu.VMEM((CPW, CH), jnp.int32),
            pltpu.VMEM((CPW, CH), jnp.int32),
            pltpu.VMEM((NBUF, CH, D), jnp.float32),
            pltpu.VMEM_SHARED((NPAD, D), jnp.float32),
            [pltpu.SemaphoreType.DMA] * NBUF,
        ],
    )
    def seg(table_hbm, src_hbm, dst_hbm, zeros_hbm, out_hbm,
            src_v, dst_v, rows_v, acc_sh, gsems):
        cid = lax.axis_index("c")
        sid = lax.axis_index("s")
        wid = cid * NS + sid
        pltpu.sync_copy(zeros_hbm.at[pl.ds(sid * RPT, RPT)],
                        acc_sh.at[pl.ds(sid * RPT, RPT)])
        pltpu.sync_copy(src_hbm.at[pl.ds(wid * CPW, CPW)], src_v)
        pltpu.sync_copy(dst_hbm.at[pl.ds(wid * CPW, CPW)], dst_v)
        plsc.subcore_barrier()

        for b in range(NBUF):
            pltpu.async_copy(table_hbm.at[src_v.at[b]], rows_v.at[b], gsems[b])

        def body(i, carry):
            k0 = i * NBUF
            for b in range(NBUF):
                k = k0 + b
                pltpu.make_async_copy(
                    table_hbm.at[src_v.at[k]], rows_v.at[b], gsems[b]).wait()
                pltpu.sync_copy(rows_v.at[b], acc_sh.at[dst_v.at[k]], add=True)

                @pl.when(k + NBUF < CPW)
                def _():
                    pltpu.async_copy(
                        table_hbm.at[src_v.at[k + NBUF]], rows_v.at[b],
                        gsems[b])
            return carry

        lax.fori_loop(0, CPW // NBUF, body, 0)
        plsc.subcore_barrier()
        pltpu.sync_copy(acc_sh.at[pl.ds(sid * RPT, RPT)],
                        out_hbm.at[pl.ds(cid * NPAD + sid * RPT, RPT)])

    return seg


_seg8 = _make_seg_kernel(8)
_seg64 = _make_seg_kernel(64)


def _dense_a(S2_ref, xp_ref, W1_ref, r1_ref, b1_ref, W2_ref, rt2_ref, b2_ref,
             m1_ref, m2_ref):
    S = S2_ref[0] + S2_ref[1]
    cnt = jnp.maximum(S[:, 1:2], 1.0)
    a = S[:, 0:1] / cnt
    h1 = jnp.maximum(
        a * W1_ref[...] + xp_ref[...] * r1_ref[...] + b1_ref[...], 0.0)
    m1_ref[...] = jnp.dot(h1, W2_ref[...], preferred_element_type=jnp.float32)
    m2_ref[...] = (jnp.dot(h1, rt2_ref[...], preferred_element_type=jnp.float32)
                   + b2_ref[...])


def _dense_b(G_ref, S2_ref, m2_ref, Wl_ref, bl_ref, z_ref):
    S = S2_ref[0] + S2_ref[1]
    invc = 1.0 / jnp.maximum(S[:, 1:2], 1.0)
    h2 = jnp.maximum((G_ref[0] + G_ref[1]) * invc + m2_ref[...], 0.0)
    z_ref[...] = (jnp.dot(h2, Wl_ref[...], preferred_element_type=jnp.float32)
                  + bl_ref[...])


def _make_decode_kernel():
    mesh = plsc.VectorSubcoreMesh(core_axis_name="c", subcore_axis_name="s")

    @functools.partial(
        pl.kernel,
        mesh=mesh,
        out_type=jax.ShapeDtypeStruct((ED * 4,), jnp.float32),
        compiler_params=pltpu.CompilerParams(needs_layout_passes=False),
        scratch_types=[
            pltpu.VMEM((NPAD * 4,), jnp.float32),
            pltpu.VMEM((DCH,), jnp.int32),
            pltpu.VMEM((DCH,), jnp.int32),
            pltpu.VMEM((DCH * 4,), jnp.float32),
        ],
    )
    def dec(zf_hbm, pos_hbm, neg_hbm, out_hbm, z_v, e0_v, e1_v, out_v):
        cid = lax.axis_index("c")
        sid = lax.axis_index("s")
        wid = cid * NS + sid
        lw = lax.rem(wid, NS)
        pltpu.sync_copy(zf_hbm, z_v)

        def run(eref):
            def chunk(c, carry):
                base = lw * EPW + c * DCH
                pltpu.sync_copy(eref.at[pl.ds(base, DCH)], e0_v)
                pltpu.sync_copy(eref.at[pl.ds(E + base, DCH)], e1_v)

                @plsc.parallel_loop(0, NG, 1, unroll=5)
                def grp(g):
                    i0 = e0_v[pl.ds(g * 16, 16)] * 4
                    i1 = e1_v[pl.ds(g * 16, 16)] * 4
                    l = []
                    for j in range(4):
                        aj = plsc.load_gather(z_v, [i0 + j])
                        bj = plsc.load_gather(z_v, [i1 + j])
                        l.append(aj * bj)
                    m = jnp.maximum(jnp.maximum(l[0], l[1]),
                                    jnp.maximum(l[2], l[3]))
                    e = [jnp.exp(v - m) for v in l]
                    r = 1.0 / ((e[0] + e[1]) + (e[2] + e[3]))
                    lane = g * 16 + lax.iota(jnp.int32, 16)
                    for j in range(4):
                        plsc.store_scatter(out_v, [j * DCH + lane], e[j] * r)
                obase = wid * EPW + c * DCH
                for j in range(4):
                    pltpu.sync_copy(out_v.at[pl.ds(j * DCH, DCH)],
                                    out_hbm.at[pl.ds(j * ED + obase, DCH)])
                return carry

            lax.fori_loop(0, EPW // DCH, chunk, 0)

        @pl.when(wid < NS)
        def _():
            run(pos_hbm)

        @pl.when(wid >= NS)
        def _():
            run(neg_hbm)

    return dec


_decode = _make_decode_kernel()


def kernel(x, train_edge_index, train_pos_edge_index, negative_edge_index,
           W1, root1, b1, W2, root2, b2, Wlin, blin):
    f32 = jnp.float32
    xp = jnp.pad(x.astype(f32), ((0, NPAD - N), (0, 0)))
    T = jnp.concatenate(
        [xp, jnp.ones((NPAD, 1), f32), jnp.zeros((NPAD, 6), f32)], axis=1)
    src2d = train_edge_index[0].reshape(NROWS, CH)
    dst2d = train_edge_index[1].reshape(NROWS, CH)
    z8 = jnp.zeros((NPAD, 8), f32)
    z64 = jnp.zeros((NPAD, 64), f32)

    S2 = _seg8(T, src2d, dst2d, z8).reshape(NC, NPAD, 8)

    m1, m2 = pl.pallas_call(
        _dense_a,
        out_shape=[jax.ShapeDtypeStruct((NPAD, 64), f32),
                   jax.ShapeDtypeStruct((NPAD, 64), f32)],
    )(S2, xp, W1, root1, b1.reshape(1, 128), W2, root2, b2.reshape(1, 64))

    G = _seg64(m1, src2d, dst2d, z64).reshape(NC, NPAD, 64)

    z = pl.pallas_call(
        _dense_b,
        out_shape=jax.ShapeDtypeStruct((NPAD, 4), f32),
    )(G, S2, m2, Wlin, blin.reshape(1, 4))

    return _decode(z.reshape(-1), train_pos_edge_index.reshape(-1),
                   negative_edge_index.reshape(-1)).reshape(4, ED).T


# speedup vs baseline: 1.0300x; 1.0300x over previous
import functools

import jax
import jax.numpy as jnp
from jax import lax
from jax.experimental import pallas as pl
from jax.experimental.pallas import tpu as pltpu
from jax.experimental.pallas import tpu_sc as plsc

N = 10000
NPAD = 10240
E = 320000
ED = 640000
NC = 2
NS = 16
NW = NC * NS
CH = 250
NROWS = E // CH
CPW = NROWS // NW
RPT = NPAD // NS

EPW = ED // NW
DCH = 2000
NG = DCH // 16
NBUF = 4


def _make_seg_kernel(D):
    mesh = plsc.VectorSubcoreMesh(core_axis_name="c", subcore_axis_name="s")

    @functools.partial(
        pl.kernel,
        mesh=mesh,
        out_type=jax.ShapeDtypeStruct((NC * NPAD, D), jnp.float32),
        compiler_params=pltpu.CompilerParams(use_tc_tiling_on_sc=False),
        scratch_types=[
            pltpu.VMEM((CPW, CH), jnp.int32),
            pltpu.VMEM((CPW, CH), jnp.int32),
            pltpu.VMEM((NBUF, CH, D), jnp.float32),
            pltpu.VMEM_SHARED((NPAD, D), jnp.float32),
            [pltpu.SemaphoreType.DMA] * NBUF,
        ],
    )
    def seg(table_hbm, src_hbm, dst_hbm, zeros_hbm, out_hbm,
            src_v, dst_v, rows_v, acc_sh, gsems):
        cid = lax.axis_index("c")
        sid = lax.axis_index("s")
        wid = cid * NS + sid
        pltpu.sync_copy(zeros_hbm.at[pl.ds(sid * RPT, RPT)],
                        acc_sh.at[pl.ds(sid * RPT, RPT)])
        pltpu.sync_copy(src_hbm.at[pl.ds(wid * CPW, CPW)], src_v)
        pltpu.sync_copy(dst_hbm.at[pl.ds(wid * CPW, CPW)], dst_v)
        plsc.subcore_barrier()

        for b in range(NBUF):
            pltpu.async_copy(table_hbm.at[src_v.at[b]], rows_v.at[b], gsems[b])

        def body(i, carry):
            k0 = i * NBUF
            for b in range(NBUF):
                k = k0 + b
                pltpu.make_async_copy(
                    table_hbm.at[src_v.at[k]], rows_v.at[b], gsems[b]).wait()
                pltpu.sync_copy(rows_v.at[b], acc_sh.at[dst_v.at[k]], add=True)

                @pl.when(k + NBUF < CPW)
                def _():
                    pltpu.async_copy(
                        table_hbm.at[src_v.at[k + NBUF]], rows_v.at[b],
                        gsems[b])
            return carry

        lax.fori_loop(0, CPW // NBUF, body, 0)
        plsc.subcore_barrier()
        pltpu.sync_copy(acc_sh.at[pl.ds(sid * RPT, RPT)],
                        out_hbm.at[pl.ds(cid * NPAD + sid * RPT, RPT)])

    return seg


_seg8 = _make_seg_kernel(8)
_seg64 = _make_seg_kernel(64)


def _dense_a(S2_ref, xp_ref, W1_ref, r1_ref, b1_ref, W2_ref, rt2_ref, b2_ref,
             m1_ref, m2_ref):
    S = S2_ref[0] + S2_ref[1]
    cnt = jnp.maximum(S[:, 1:2], 1.0)
    a = S[:, 0:1] / cnt
    h1 = jnp.maximum(
        a * W1_ref[...] + xp_ref[...] * r1_ref[...] + b1_ref[...], 0.0)
    m1_ref[...] = jnp.dot(h1, W2_ref[...], preferred_element_type=jnp.float32)
    m2_ref[...] = (jnp.dot(h1, rt2_ref[...], preferred_element_type=jnp.float32)
                   + b2_ref[...])


def _dense_b(G_ref, S2_ref, m2_ref, Wl_ref, bl_ref, z_ref):
    S = S2_ref[0] + S2_ref[1]
    invc = 1.0 / jnp.maximum(S[:, 1:2], 1.0)
    h2 = jnp.maximum((G_ref[0] + G_ref[1]) * invc + m2_ref[...], 0.0)
    z_ref[...] = (jnp.dot(h2, Wl_ref[...], preferred_element_type=jnp.float32)
                  + bl_ref[...])


def _make_decode_kernel():
    mesh = plsc.VectorSubcoreMesh(core_axis_name="c", subcore_axis_name="s")

    @functools.partial(
        pl.kernel,
        mesh=mesh,
        out_type=jax.ShapeDtypeStruct((ED * 4,), jnp.float32),
        compiler_params=pltpu.CompilerParams(needs_layout_passes=False),
        scratch_types=[
            pltpu.VMEM((NPAD * 4,), jnp.float32),
            pltpu.VMEM((DCH,), jnp.int32),
            pltpu.VMEM((DCH,), jnp.int32),
            pltpu.VMEM((DCH * 4,), jnp.float32),
        ],
    )
    def dec(zf_hbm, pos_hbm, neg_hbm, out_hbm, z_v, e0_v, e1_v, out_v):
        cid = lax.axis_index("c")
        sid = lax.axis_index("s")
        wid = cid * NS + sid
        lw = lax.rem(wid, NS)
        pltpu.sync_copy(zf_hbm, z_v)

        def run(eref):
            def chunk(c, carry):
                base = lw * EPW + c * DCH
                pltpu.sync_copy(eref.at[pl.ds(base, DCH)], e0_v)
                pltpu.sync_copy(eref.at[pl.ds(E + base, DCH)], e1_v)

                @plsc.parallel_loop(0, NG, 1, unroll=4)
                def grp(g):
                    i0 = e0_v[pl.ds(g * 16, 16)] * 4
                    i1 = e1_v[pl.ds(g * 16, 16)] * 4
                    l = []
                    for j in range(4):
                        aj = plsc.load_gather(z_v, [i0 + j])
                        bj = plsc.load_gather(z_v, [i1 + j])
                        l.append(aj * bj)
                    m = jnp.maximum(jnp.maximum(l[0], l[1]),
                                    jnp.maximum(l[2], l[3]))
                    e = [jnp.exp(v - m) for v in l]
                    r = 1.0 / ((e[0] + e[1]) + (e[2] + e[3]))
                    lane = g * 16 + lax.iota(jnp.int32, 16)
                    for j in range(4):
                        plsc.store_scatter(out_v, [j * DCH + lane], e[j] * r)
                obase = wid * EPW + c * DCH
                for j in range(4):
                    pltpu.sync_copy(out_v.at[pl.ds(j * DCH, DCH)],
                                    out_hbm.at[pl.ds(j * ED + obase, DCH)])
                return carry

            lax.fori_loop(0, EPW // DCH, chunk, 0)

        @pl.when(wid < NS)
        def _():
            run(pos_hbm)

        @pl.when(wid >= NS)
        def _():
            run(neg_hbm)

    return dec


_decode = _make_decode_kernel()


def kernel(x, train_edge_index, train_pos_edge_index, negative_edge_index,
           W1, root1, b1, W2, root2, b2, Wlin, blin):
    f32 = jnp.float32
    xp = jnp.pad(x.astype(f32), ((0, NPAD - N), (0, 0)))
    T = jnp.concatenate(
        [xp, jnp.ones((NPAD, 1), f32), jnp.zeros((NPAD, 6), f32)], axis=1)
    src2d = train_edge_index[0].reshape(NROWS, CH)
    dst2d = train_edge_index[1].reshape(NROWS, CH)
    z8 = jnp.zeros((NPAD, 8), f32)
    z64 = jnp.zeros((NPAD, 64), f32)

    S2 = _seg8(T, src2d, dst2d, z8).reshape(NC, NPAD, 8)

    m1, m2 = pl.pallas_call(
        _dense_a,
        out_shape=[jax.ShapeDtypeStruct((NPAD, 64), f32),
                   jax.ShapeDtypeStruct((NPAD, 64), f32)],
    )(S2, xp, W1, root1, b1.reshape(1, 128), W2, root2, b2.reshape(1, 64))

    G = _seg64(m1, src2d, dst2d, z64).reshape(NC, NPAD, 64)

    z = pl.pallas_call(
        _dense_b,
        out_shape=jax.ShapeDtypeStruct((NPAD, 4), f32),
    )(G, S2, m2, Wlin, blin.reshape(1, 4))

    return _decode(z.reshape(-1), train_pos_edge_index.reshape(-1),
                   negative_edge_index.reshape(-1)).reshape(4, ED).T
